# Initial kernel scaffold; baseline (speedup 1.0000x reference)
import jax
import jax.numpy as jnp
from jax.experimental import pallas as pl


def kernel(x, edge_index, graph_ids, W_init, W1, b1, Wr1, br1, W2, b2, Wr2, br2, W3, b3, Wr3, br3):
    raise NotImplementedError("write your pallas kernel here")


# speedup vs baseline: 5.3851x; 5.3851x over previous
import functools

import jax
import jax.numpy as jnp
from jax import lax
from jax.experimental import pallas as pl
from jax.experimental.pallas import tpu as pltpu
from jax.experimental.pallas import tpu_sc as plsc

N = 10000
E = 320000
D = 128
G = 256

NC = 2
NS = 16
NW = NC * NS

NPAD = 10240
GRP = 128
KE = 80
EW = KE * GRP
E2 = NW * EW
NODE_W = NPAD // NW
SEG = 384
SEG_T = SEG // NS
ROWS_T = NPAD // NS

_mesh = plsc.VectorSubcoreMesh(
    core_axis_name="c", subcore_axis_name="s", num_cores=NC, num_subcores=NS)
_sc_params = pltpu.CompilerParams(needs_layout_passes=False)


@functools.partial(
    pl.kernel,
    out_type=[
        jax.ShapeDtypeStruct((NW * NPAD,), jnp.float32),
        jax.ShapeDtypeStruct((NW * NPAD,), jnp.float32),
        jax.ShapeDtypeStruct((NW * SEG,), jnp.float32),
    ],
    mesh=_mesh,
    scratch_types=[
        pltpu.VMEM((EW,), jnp.int32),
        pltpu.VMEM((EW,), jnp.int32),
        pltpu.VMEM((NODE_W,), jnp.int32),
        pltpu.VMEM((NPAD,), jnp.float32),
        pltpu.VMEM((NPAD,), jnp.float32),
        pltpu.VMEM((SEG,), jnp.float32),
    ],
    compiler_params=_sc_params,
)
def _degrees(src_hbm, dst_hbm, gid_hbm, od_hbm, id_hbm, cnt_hbm,
             srcv, dstv, gidv, oacc, iacc, cacc):
    cid = lax.axis_index("c")
    sid = lax.axis_index("s")
    wid = sid * NC + cid
    pltpu.sync_copy(src_hbm.at[pl.ds(wid * EW, EW)], srcv)
    pltpu.sync_copy(dst_hbm.at[pl.ds(wid * EW, EW)], dstv)
    pltpu.sync_copy(gid_hbm.at[pl.ds(wid * NODE_W, NODE_W)], gidv)
    zeros16 = jnp.zeros((16,), jnp.float32)
    ones16 = jnp.ones((16,), jnp.float32)

    def zero_nodes(i, c):
        oacc[pl.ds(i * 16, 16)] = zeros16
        iacc[pl.ds(i * 16, 16)] = zeros16
        return c

    lax.fori_loop(0, NPAD // 16, zero_nodes, 0)

    def zero_cnt(i, c):
        cacc[pl.ds(i * 16, 16)] = zeros16
        return c

    lax.fori_loop(0, SEG // 16, zero_cnt, 0)

    def edge_step(i, c):
        si = srcv[pl.ds(i * 16, 16)]
        plsc.addupdate_scatter(oacc, [si], ones16)
        di = dstv[pl.ds(i * 16, 16)]
        plsc.addupdate_scatter(iacc, [di], ones16)
        return c

    lax.fori_loop(0, EW // 16, edge_step, 0)

    def gid_step(i, c):
        gi = gidv[pl.ds(i * 16, 16)]
        plsc.addupdate_scatter(cacc, [gi], ones16)
        return c

    lax.fori_loop(0, NODE_W // 16, gid_step, 0)

    pltpu.sync_copy(oacc, od_hbm.at[pl.ds(wid * NPAD, NPAD)])
    pltpu.sync_copy(iacc, id_hbm.at[pl.ds(wid * NPAD, NPAD)])
    pltpu.sync_copy(cacc, cnt_hbm.at[pl.ds(wid * SEG, SEG)])


NH = NPAD // NC
NHP = 5248
ROWS_T2 = NHP // NS
KE2 = E2 // (NS * GRP)


@functools.partial(
    pl.kernel,
    out_type=jax.ShapeDtypeStruct((NC, NHP, D), jnp.float32),
    mesh=_mesh,
    scratch_types=[
        pltpu.VMEM((KE2, GRP), jnp.int32),
        pltpu.VMEM((KE2, GRP), jnp.int32),
        pltpu.VMEM((GRP, D), jnp.float32),
        pltpu.VMEM((GRP, D), jnp.float32),
        pltpu.VMEM_SHARED((NHP, D), jnp.float32),
        pltpu.SemaphoreType.DMA,
        pltpu.SemaphoreType.DMA,
    ],
    compiler_params=_sc_params,
)
def _aggregate(m_hbm, src_hbm, dst_hbm, out_hbm,
               srcv, dstv, buf_a, buf_b, sh_acc, sem_a, sem_b):
    cid = lax.axis_index("c")
    sid = lax.axis_index("s")
    pltpu.sync_copy(src_hbm.at[sid], srcv)
    pltpu.sync_copy(dst_hbm.at[cid, sid], dstv)

    zeros16 = jnp.zeros((16,), jnp.float32)

    def zero_buf(i, c):
        buf_a[i // 8, pl.ds((i % 8) * 16, 16)] = zeros16
        return c

    lax.fori_loop(0, GRP * D // 16, zero_buf, 0)
    base = sid * ROWS_T2
    for off, sz in ((0, GRP), (GRP, GRP), (2 * GRP, ROWS_T2 - 2 * GRP)):
        pltpu.sync_copy(buf_a.at[pl.ds(0, sz)],
                        sh_acc.at[pl.ds(base + off, sz)])
    plsc.subcore_barrier()

    def pair_step(p, c):
        j = p * 2
        cp0 = pltpu.async_copy(m_hbm.at[srcv.at[j]], buf_a, sem_a)
        cp1 = pltpu.async_copy(m_hbm.at[srcv.at[j + 1]], buf_b, sem_b)
        cp0.wait()
        pltpu.sync_copy(buf_a, sh_acc.at[dstv.at[j]], add=True)
        cp1.wait()
        pltpu.sync_copy(buf_b, sh_acc.at[dstv.at[j + 1]], add=True)
        return c

    lax.fori_loop(0, KE2 // 2, pair_step, 0)
    plsc.subcore_barrier()
    for off, sz in ((0, GRP), (GRP, GRP), (2 * GRP, ROWS_T2 - 2 * GRP)):
        pltpu.sync_copy(sh_acc.at[pl.ds(base + off, sz)],
                        out_hbm.at[cid, pl.ds(base + off, sz)])


@functools.partial(
    pl.kernel,
    out_type=jax.ShapeDtypeStruct((NC, SEG, D), jnp.float32),
    mesh=_mesh,
    scratch_types=[
        pltpu.VMEM((NODE_W,), jnp.int32),
        pltpu.VMEM((NODE_W, D), jnp.float32),
        pltpu.VMEM_SHARED((SEG, D), jnp.float32),
        pltpu.SemaphoreType.DMA,
    ],
    compiler_params=_sc_params,
)
def _pool(h_hbm, gid_hbm, out_hbm, gidv, buf, sh_seg, sem):
    cid = lax.axis_index("c")
    sid = lax.axis_index("s")
    wid = sid * NC + cid
    pltpu.sync_copy(gid_hbm.at[pl.ds(wid * NODE_W, NODE_W)], gidv)
    zeros16 = jnp.zeros((16,), jnp.float32)

    def zero_buf(i, c):
        buf[i // 8, pl.ds((i % 8) * 16, 16)] = zeros16
        return c

    lax.fori_loop(0, SEG_T * D // 16, zero_buf, 0)
    pltpu.sync_copy(buf.at[pl.ds(0, SEG_T)], sh_seg.at[pl.ds(sid * SEG_T, SEG_T)])
    plsc.subcore_barrier()
    pltpu.async_copy(h_hbm.at[pl.ds(wid * NODE_W, NODE_W)], buf, sem).wait()

    def seg_step(k, c):
        gi = gidv[pl.ds(k * 16, 16)]
        pltpu.sync_copy(buf.at[pl.ds(k * 16, 16)], sh_seg.at[gi], add=True)
        return c

    lax.fori_loop(0, NODE_W // 16, seg_step, 0)
    plsc.subcore_barrier()
    pltpu.sync_copy(sh_seg.at[pl.ds(sid * SEG_T, SEG_T)],
                    out_hbm.at[cid, pl.ds(sid * SEG_T, SEG_T)])


RB = 1024


def _embed_body(x_ref, od_ref, id_ref, wi_ref, w1_ref, wr1_ref, br1_ref,
                m_ref, r_ref, on_ref, in_ref):
    od = jnp.clip(jnp.sum(od_ref[...], axis=0), 1.0, None)
    idg = jnp.clip(jnp.sum(id_ref[...], axis=0), 1.0, None)
    onorm = lax.rsqrt(od)[:, None]
    inorm = lax.rsqrt(idg)[:, None]
    on_ref[...] = onorm
    in_ref[...] = inorm
    h = jnp.dot(x_ref[...], wi_ref[...], preferred_element_type=jnp.float32)
    m_ref[...] = jnp.dot(h * onorm, w1_ref[...], preferred_element_type=jnp.float32)
    r_ref[...] = jax.nn.relu(
        jnp.dot(h, wr1_ref[...], preferred_element_type=jnp.float32) + br1_ref[...])


def _embed(xp, odp, idp, wi, w1, wr1, br1):
    grid = NPAD // RB
    return pl.pallas_call(
        _embed_body,
        grid=(grid,),
        in_specs=[
            pl.BlockSpec((RB, D), lambda i: (i, 0)),
            pl.BlockSpec((NW, RB), lambda i: (0, i)),
            pl.BlockSpec((NW, RB), lambda i: (0, i)),
            pl.BlockSpec((D, D), lambda i: (0, 0)),
            pl.BlockSpec((D, D), lambda i: (0, 0)),
            pl.BlockSpec((D, D), lambda i: (0, 0)),
            pl.BlockSpec((1, D), lambda i: (0, 0)),
        ],
        out_specs=[
            pl.BlockSpec((RB, D), lambda i: (i, 0)),
            pl.BlockSpec((RB, D), lambda i: (i, 0)),
            pl.BlockSpec((RB, 1), lambda i: (i, 0)),
            pl.BlockSpec((RB, 1), lambda i: (i, 0)),
        ],
        out_shape=[
            jax.ShapeDtypeStruct((NPAD, D), jnp.float32),
            jax.ShapeDtypeStruct((NPAD, D), jnp.float32),
            jax.ShapeDtypeStruct((NPAD, 1), jnp.float32),
            jax.ShapeDtypeStruct((NPAD, 1), jnp.float32),
        ],
    )(xp, odp, idp, wi, w1, wr1, br1)


def _layer_body(agg_ref, in_ref, b_ref, r_ref, on_ref, w_ref, wr_ref, br_ref,
                m_ref, rn_ref):
    agg = agg_ref[0]
    h = jax.nn.relu(agg * in_ref[...] + b_ref[...]) + r_ref[...]
    m_ref[...] = jnp.dot(h * on_ref[...], w_ref[...],
                         preferred_element_type=jnp.float32)
    rn_ref[...] = jax.nn.relu(
        jnp.dot(h, wr_ref[...], preferred_element_type=jnp.float32) + br_ref[...])


def _layer_update(aggp, inorm, b, r, onorm, w_next, wr_next, br_next):
    grid = NPAD // RB
    return pl.pallas_call(
        _layer_body,
        grid=(grid,),
        in_specs=[
            pl.BlockSpec((1, RB, D), lambda i: (i // 5, i % 5, 0)),
            pl.BlockSpec((RB, 1), lambda i: (i, 0)),
            pl.BlockSpec((1, D), lambda i: (0, 0)),
            pl.BlockSpec((RB, D), lambda i: (i, 0)),
            pl.BlockSpec((RB, 1), lambda i: (i, 0)),
            pl.BlockSpec((D, D), lambda i: (0, 0)),
            pl.BlockSpec((D, D), lambda i: (0, 0)),
            pl.BlockSpec((1, D), lambda i: (0, 0)),
        ],
        out_specs=[
            pl.BlockSpec((RB, D), lambda i: (i, 0)),
            pl.BlockSpec((RB, D), lambda i: (i, 0)),
        ],
        out_shape=[
            jax.ShapeDtypeStruct((NPAD, D), jnp.float32),
            jax.ShapeDtypeStruct((NPAD, D), jnp.float32),
        ],
    )(aggp, inorm, b, r, onorm, w_next, wr_next, br_next)


def _final_body(agg_ref, in_ref, b_ref, r_ref, h_ref):
    agg = agg_ref[0]
    h_ref[...] = jax.nn.relu(agg * in_ref[...] + b_ref[...]) + r_ref[...]


def _final_h(aggp, inorm, b, r):
    grid = NPAD // RB
    return pl.pallas_call(
        _final_body,
        grid=(grid,),
        in_specs=[
            pl.BlockSpec((1, RB, D), lambda i: (i // 5, i % 5, 0)),
            pl.BlockSpec((RB, 1), lambda i: (i, 0)),
            pl.BlockSpec((1, D), lambda i: (0, 0)),
            pl.BlockSpec((RB, D), lambda i: (i, 0)),
        ],
        out_specs=pl.BlockSpec((RB, D), lambda i: (i, 0)),
        out_shape=jax.ShapeDtypeStruct((NPAD, D), jnp.float32),
    )(aggp, inorm, b, r)


def _divide_body(s_ref, c_ref, o_ref):
    s = s_ref[0] + s_ref[1]
    c = jnp.clip(jnp.sum(c_ref[...], axis=0), 1.0, None)
    o_ref[...] = s / c[:, None]


def _divide(sums, cnts):
    return pl.pallas_call(
        _divide_body,
        grid=(1,),
        in_specs=[
            pl.BlockSpec((NC, G, D), lambda i: (0, 0, 0)),
            pl.BlockSpec((NW, G), lambda i: (0, 0)),
        ],
        out_specs=pl.BlockSpec((G, D), lambda i: (0, 0)),
        out_shape=jax.ShapeDtypeStruct((G, D), jnp.float32),
    )(sums, cnts)


def kernel(x, edge_index, graph_ids, W_init,
           W1, b1, Wr1, br1, W2, b2, Wr2, br2, W3, b3, Wr3, br3):
    npad_ids = N + (jnp.arange(E2 - E, dtype=jnp.int32) % (NPAD - N))
    src = jnp.concatenate([edge_index[0], npad_ids])
    dst = jnp.concatenate([edge_index[1], npad_ids])
    src3 = src.reshape(NS, KE2, GRP)
    trash = NH + (jnp.arange(E2, dtype=jnp.int32) % (NHP - NH))
    dst_c0 = jnp.where(dst < NH, dst, trash)
    dst_c1 = jnp.where(dst >= NH, dst - NH, trash)
    dst4 = jnp.stack([dst_c0.reshape(NS, KE2, GRP),
                      dst_c1.reshape(NS, KE2, GRP)], axis=0)
    gid_pad = jnp.concatenate(
        [graph_ids, jnp.full((NPAD - N,), G, dtype=jnp.int32)])
    xp = jnp.zeros((NPAD, D), jnp.float32).at[:N].set(x)

    odp, idp, cntp = _degrees(src, dst, gid_pad)
    odp = odp.reshape(NW, NPAD)
    idp = idp.reshape(NW, NPAD)
    cntp = cntp.reshape(NW, SEG)

    m1, r1, onorm, inorm = _embed(xp, odp, idp, W_init, W1, Wr1,
                                  br1.reshape(1, D))

    agg1 = _aggregate(m1, src3, dst4)
    m2, r2 = _layer_update(agg1, inorm, b1.reshape(1, D), r1, onorm, W2, Wr2,
                           br2.reshape(1, D))
    agg2 = _aggregate(m2, src3, dst4)
    m3, r3 = _layer_update(agg2, inorm, b2.reshape(1, D), r2, onorm, W3, Wr3,
                           br3.reshape(1, D))
    agg3 = _aggregate(m3, src3, dst4)
    h3 = _final_h(agg3, inorm, b3.reshape(1, D), r3)

    sums = _pool(h3, gid_pad)
    return _divide(sums[:, :G, :], cntp[:, :G])


# speedup vs baseline: 5.5180x; 1.0247x over previous
import functools

import jax
import jax.numpy as jnp
from jax import lax
from jax.experimental import pallas as pl
from jax.experimental.pallas import tpu as pltpu
from jax.experimental.pallas import tpu_sc as plsc

N = 10000
E = 320000
D = 128
G = 256

NC = 2
NS = 16
NW = NC * NS

NPAD = 10240
GRP = 128
KE = 80
EW = KE * GRP
E2 = NW * EW
NODE_W = NPAD // NW
SEG = 384
SEG_T = SEG // NS
ROWS_T = NPAD // NS

_mesh = plsc.VectorSubcoreMesh(
    core_axis_name="c", subcore_axis_name="s", num_cores=NC, num_subcores=NS)
_sc_params = pltpu.CompilerParams(needs_layout_passes=False)


@functools.partial(
    pl.kernel,
    out_type=[
        jax.ShapeDtypeStruct((NW * NPAD,), jnp.float32),
        jax.ShapeDtypeStruct((NW * NPAD,), jnp.float32),
        jax.ShapeDtypeStruct((NW * SEG,), jnp.float32),
    ],
    mesh=_mesh,
    scratch_types=[
        pltpu.VMEM((EW,), jnp.int32),
        pltpu.VMEM((EW,), jnp.int32),
        pltpu.VMEM((NODE_W,), jnp.int32),
        pltpu.VMEM((NPAD,), jnp.float32),
        pltpu.VMEM((NPAD,), jnp.float32),
        pltpu.VMEM((SEG,), jnp.float32),
    ],
    compiler_params=_sc_params,
)
def _degrees(src_hbm, dst_hbm, gid_hbm, od_hbm, id_hbm, cnt_hbm,
             srcv, dstv, gidv, oacc, iacc, cacc):
    cid = lax.axis_index("c")
    sid = lax.axis_index("s")
    wid = sid * NC + cid
    pltpu.sync_copy(src_hbm.at[pl.ds(wid * EW, EW)], srcv)
    pltpu.sync_copy(dst_hbm.at[pl.ds(wid * EW, EW)], dstv)
    pltpu.sync_copy(gid_hbm.at[pl.ds(wid * NODE_W, NODE_W)], gidv)
    zeros16 = jnp.zeros((16,), jnp.float32)
    ones16 = jnp.ones((16,), jnp.float32)

    def zero_nodes(i, c):
        oacc[pl.ds(i * 16, 16)] = zeros16
        iacc[pl.ds(i * 16, 16)] = zeros16
        return c

    lax.fori_loop(0, NPAD // 16, zero_nodes, 0)

    def zero_cnt(i, c):
        cacc[pl.ds(i * 16, 16)] = zeros16
        return c

    lax.fori_loop(0, SEG // 16, zero_cnt, 0)

    def edge_step(i, c):
        si = srcv[pl.ds(i * 16, 16)]
        plsc.addupdate_scatter(oacc, [si], ones16)
        di = dstv[pl.ds(i * 16, 16)]
        plsc.addupdate_scatter(iacc, [di], ones16)
        return c

    lax.fori_loop(0, EW // 16, edge_step, 0)

    def gid_step(i, c):
        gi = gidv[pl.ds(i * 16, 16)]
        plsc.addupdate_scatter(cacc, [gi], ones16)
        return c

    lax.fori_loop(0, NODE_W // 16, gid_step, 0)

    pltpu.sync_copy(oacc, od_hbm.at[pl.ds(wid * NPAD, NPAD)])
    pltpu.sync_copy(iacc, id_hbm.at[pl.ds(wid * NPAD, NPAD)])
    pltpu.sync_copy(cacc, cnt_hbm.at[pl.ds(wid * SEG, SEG)])


NH = NPAD // NC
NHP = 5248
ROWS_T2 = NHP // NS
KE2 = E2 // (NS * GRP)


@functools.partial(
    pl.kernel,
    out_type=jax.ShapeDtypeStruct((NC, NHP, D), jnp.float32),
    mesh=_mesh,
    scratch_types=[
        pltpu.VMEM((KE2, GRP), jnp.int32),
        pltpu.VMEM((KE2, GRP), jnp.int32),
        pltpu.VMEM((GRP, D), jnp.float32),
        pltpu.VMEM((GRP, D), jnp.float32),
        pltpu.VMEM_SHARED((NHP, D), jnp.float32),
        pltpu.SemaphoreType.DMA,
        pltpu.SemaphoreType.DMA,
        pltpu.SemaphoreType.DMA,
        pltpu.SemaphoreType.DMA,
    ],
    compiler_params=_sc_params,
)
def _aggregate(m_hbm, src_hbm, dst_hbm, out_hbm,
               srcv, dstv, b0, b1, sh_acc,
               g0, g1, s0, s1):
    cid = lax.axis_index("c")
    sid = lax.axis_index("s")
    bufs = (b0, b1)
    gsem = (g0, g1)
    ssem = (s0, s1)
    NB = 2
    nblk = KE2 // NB
    pltpu.sync_copy(src_hbm.at[sid], srcv)
    pltpu.sync_copy(dst_hbm.at[cid, sid], dstv)

    zeros16 = jnp.zeros((16,), jnp.float32)

    def zero_buf(i, c):
        b0[i // 8, pl.ds((i % 8) * 16, 16)] = zeros16
        return c

    lax.fori_loop(0, GRP * D // 16, zero_buf, 0)
    base = sid * ROWS_T2
    for off, sz in ((0, GRP), (GRP, GRP), (2 * GRP, ROWS_T2 - 2 * GRP)):
        pltpu.sync_copy(b0.at[pl.ds(0, sz)],
                        sh_acc.at[pl.ds(base + off, sz)])
    plsc.subcore_barrier()

    for q in range(NB):
        pltpu.async_copy(m_hbm.at[srcv.at[q]], bufs[q], gsem[q])

    def blk(p, c):
        j = p * NB
        for q in range(NB):
            pltpu.make_async_copy(m_hbm.at[srcv.at[j + q]],
                                  bufs[q], gsem[q]).wait()
            pltpu.async_copy(bufs[q], sh_acc.at[dstv.at[j + q]], ssem[q],
                             add=True)

        @pl.when(p + 1 < nblk)
        def _():
            for q in range(NB):
                pltpu.make_async_copy(bufs[q], sh_acc.at[dstv.at[j + q]],
                                      ssem[q]).wait()
                pltpu.async_copy(m_hbm.at[srcv.at[j + NB + q]],
                                 bufs[q], gsem[q])
        return c

    lax.fori_loop(0, nblk, blk, 0)
    for q in range(NB):
        pltpu.make_async_copy(bufs[q], sh_acc.at[dstv.at[KE2 - NB + q]],
                              ssem[q]).wait()
    plsc.subcore_barrier()
    for off, sz in ((0, GRP), (GRP, GRP), (2 * GRP, ROWS_T2 - 2 * GRP)):
        pltpu.sync_copy(sh_acc.at[pl.ds(base + off, sz)],
                        out_hbm.at[cid, pl.ds(base + off, sz)])


@functools.partial(
    pl.kernel,
    out_type=jax.ShapeDtypeStruct((NC, SEG, D), jnp.float32),
    mesh=_mesh,
    scratch_types=[
        pltpu.VMEM((NODE_W,), jnp.int32),
        pltpu.VMEM((NODE_W, D), jnp.float32),
        pltpu.VMEM_SHARED((SEG, D), jnp.float32),
        pltpu.SemaphoreType.DMA,
    ],
    compiler_params=_sc_params,
)
def _pool(h_hbm, gid_hbm, out_hbm, gidv, buf, sh_seg, sem):
    cid = lax.axis_index("c")
    sid = lax.axis_index("s")
    wid = sid * NC + cid
    pltpu.sync_copy(gid_hbm.at[pl.ds(wid * NODE_W, NODE_W)], gidv)
    zeros16 = jnp.zeros((16,), jnp.float32)

    def zero_buf(i, c):
        buf[i // 8, pl.ds((i % 8) * 16, 16)] = zeros16
        return c

    lax.fori_loop(0, SEG_T * D // 16, zero_buf, 0)
    pltpu.sync_copy(buf.at[pl.ds(0, SEG_T)], sh_seg.at[pl.ds(sid * SEG_T, SEG_T)])
    plsc.subcore_barrier()
    pltpu.async_copy(h_hbm.at[pl.ds(wid * NODE_W, NODE_W)], buf, sem).wait()

    def seg_step(k, c):
        gi = gidv[pl.ds(k * 16, 16)]
        pltpu.sync_copy(buf.at[pl.ds(k * 16, 16)], sh_seg.at[gi], add=True)
        return c

    lax.fori_loop(0, NODE_W // 16, seg_step, 0)
    plsc.subcore_barrier()
    pltpu.sync_copy(sh_seg.at[pl.ds(sid * SEG_T, SEG_T)],
                    out_hbm.at[cid, pl.ds(sid * SEG_T, SEG_T)])


RB = 1024


def _embed_body(x_ref, od_ref, id_ref, wi_ref, w1_ref, wr1_ref, br1_ref,
                m_ref, r_ref, on_ref, in_ref):
    od = jnp.clip(jnp.sum(od_ref[...], axis=0), 1.0, None)
    idg = jnp.clip(jnp.sum(id_ref[...], axis=0), 1.0, None)
    onorm = lax.rsqrt(od)[:, None]
    inorm = lax.rsqrt(idg)[:, None]
    on_ref[...] = onorm
    in_ref[...] = inorm
    h = jnp.dot(x_ref[...], wi_ref[...], preferred_element_type=jnp.float32)
    m_ref[...] = jnp.dot(h * onorm, w1_ref[...], preferred_element_type=jnp.float32)
    r_ref[...] = jax.nn.relu(
        jnp.dot(h, wr1_ref[...], preferred_element_type=jnp.float32) + br1_ref[...])


def _embed(xp, odp, idp, wi, w1, wr1, br1):
    grid = NPAD // RB
    return pl.pallas_call(
        _embed_body,
        grid=(grid,),
        in_specs=[
            pl.BlockSpec((RB, D), lambda i: (i, 0)),
            pl.BlockSpec((NW, RB), lambda i: (0, i)),
            pl.BlockSpec((NW, RB), lambda i: (0, i)),
            pl.BlockSpec((D, D), lambda i: (0, 0)),
            pl.BlockSpec((D, D), lambda i: (0, 0)),
            pl.BlockSpec((D, D), lambda i: (0, 0)),
            pl.BlockSpec((1, D), lambda i: (0, 0)),
        ],
        out_specs=[
            pl.BlockSpec((RB, D), lambda i: (i, 0)),
            pl.BlockSpec((RB, D), lambda i: (i, 0)),
            pl.BlockSpec((RB, 1), lambda i: (i, 0)),
            pl.BlockSpec((RB, 1), lambda i: (i, 0)),
        ],
        out_shape=[
            jax.ShapeDtypeStruct((NPAD, D), jnp.float32),
            jax.ShapeDtypeStruct((NPAD, D), jnp.float32),
            jax.ShapeDtypeStruct((NPAD, 1), jnp.float32),
            jax.ShapeDtypeStruct((NPAD, 1), jnp.float32),
        ],
    )(xp, odp, idp, wi, w1, wr1, br1)


def _layer_body(agg_ref, in_ref, b_ref, r_ref, on_ref, w_ref, wr_ref, br_ref,
                m_ref, rn_ref):
    agg = agg_ref[0]
    h = jax.nn.relu(agg * in_ref[...] + b_ref[...]) + r_ref[...]
    m_ref[...] = jnp.dot(h * on_ref[...], w_ref[...],
                         preferred_element_type=jnp.float32)
    rn_ref[...] = jax.nn.relu(
        jnp.dot(h, wr_ref[...], preferred_element_type=jnp.float32) + br_ref[...])


def _layer_update(aggp, inorm, b, r, onorm, w_next, wr_next, br_next):
    grid = NPAD // RB
    return pl.pallas_call(
        _layer_body,
        grid=(grid,),
        in_specs=[
            pl.BlockSpec((1, RB, D), lambda i: (i // 5, i % 5, 0)),
            pl.BlockSpec((RB, 1), lambda i: (i, 0)),
            pl.BlockSpec((1, D), lambda i: (0, 0)),
            pl.BlockSpec((RB, D), lambda i: (i, 0)),
            pl.BlockSpec((RB, 1), lambda i: (i, 0)),
            pl.BlockSpec((D, D), lambda i: (0, 0)),
            pl.BlockSpec((D, D), lambda i: (0, 0)),
            pl.BlockSpec((1, D), lambda i: (0, 0)),
        ],
        out_specs=[
            pl.BlockSpec((RB, D), lambda i: (i, 0)),
            pl.BlockSpec((RB, D), lambda i: (i, 0)),
        ],
        out_shape=[
            jax.ShapeDtypeStruct((NPAD, D), jnp.float32),
            jax.ShapeDtypeStruct((NPAD, D), jnp.float32),
        ],
    )(aggp, inorm, b, r, onorm, w_next, wr_next, br_next)


def _final_body(agg_ref, in_ref, b_ref, r_ref, h_ref):
    agg = agg_ref[0]
    h_ref[...] = jax.nn.relu(agg * in_ref[...] + b_ref[...]) + r_ref[...]


def _final_h(aggp, inorm, b, r):
    grid = NPAD // RB
    return pl.pallas_call(
        _final_body,
        grid=(grid,),
        in_specs=[
            pl.BlockSpec((1, RB, D), lambda i: (i // 5, i % 5, 0)),
            pl.BlockSpec((RB, 1), lambda i: (i, 0)),
            pl.BlockSpec((1, D), lambda i: (0, 0)),
            pl.BlockSpec((RB, D), lambda i: (i, 0)),
        ],
        out_specs=pl.BlockSpec((RB, D), lambda i: (i, 0)),
        out_shape=jax.ShapeDtypeStruct((NPAD, D), jnp.float32),
    )(aggp, inorm, b, r)


def _divide_body(s_ref, c_ref, o_ref):
    s = s_ref[0] + s_ref[1]
    c = jnp.clip(jnp.sum(c_ref[...], axis=0), 1.0, None)
    o_ref[...] = s / c[:, None]


def _divide(sums, cnts):
    return pl.pallas_call(
        _divide_body,
        grid=(1,),
        in_specs=[
            pl.BlockSpec((NC, G, D), lambda i: (0, 0, 0)),
            pl.BlockSpec((NW, G), lambda i: (0, 0)),
        ],
        out_specs=pl.BlockSpec((G, D), lambda i: (0, 0)),
        out_shape=jax.ShapeDtypeStruct((G, D), jnp.float32),
    )(sums, cnts)


def kernel(x, edge_index, graph_ids, W_init,
           W1, b1, Wr1, br1, W2, b2, Wr2, br2, W3, b3, Wr3, br3):
    npad_ids = N + (jnp.arange(E2 - E, dtype=jnp.int32) % (NPAD - N))
    src = jnp.concatenate([edge_index[0], npad_ids])
    dst = jnp.concatenate([edge_index[1], npad_ids])
    src3 = src.reshape(NS, KE2, GRP)
    trash = NH + (jnp.arange(E2, dtype=jnp.int32) % (NHP - NH))
    dst_c0 = jnp.where(dst < NH, dst, trash)
    dst_c1 = jnp.where(dst >= NH, dst - NH, trash)
    dst4 = jnp.stack([dst_c0.reshape(NS, KE2, GRP),
                      dst_c1.reshape(NS, KE2, GRP)], axis=0)
    gid_pad = jnp.concatenate(
        [graph_ids, jnp.full((NPAD - N,), G, dtype=jnp.int32)])
    xp = jnp.zeros((NPAD, D), jnp.float32).at[:N].set(x)

    odp, idp, cntp = _degrees(src, dst, gid_pad)
    odp = odp.reshape(NW, NPAD)
    idp = idp.reshape(NW, NPAD)
    cntp = cntp.reshape(NW, SEG)

    m1, r1, onorm, inorm = _embed(xp, odp, idp, W_init, W1, Wr1,
                                  br1.reshape(1, D))

    agg1 = _aggregate(m1, src3, dst4)
    m2, r2 = _layer_update(agg1, inorm, b1.reshape(1, D), r1, onorm, W2, Wr2,
                           br2.reshape(1, D))
    agg2 = _aggregate(m2, src3, dst4)
    m3, r3 = _layer_update(agg2, inorm, b2.reshape(1, D), r2, onorm, W3, Wr3,
                           br3.reshape(1, D))
    agg3 = _aggregate(m3, src3, dst4)
    h3 = _final_h(agg3, inorm, b3.reshape(1, D), r3)

    sums = _pool(h3, gid_pad)
    return _divide(sums[:, :G, :], cntp[:, :G])
